# baseline (speedup 1.0000x reference)
import jax
import jax.numpy as jnp
from jax.experimental import pallas as pl
from jax.experimental.pallas import tpu as pltpu

_B = 4
_D = 2048
_HIDDEN = 2048
_K = 64
_NT = 8
_TC = 2048 // _NT


def _fused_body(h_ref, w_ref, idx_ref, val_ref, acc_ref):
    j = pl.program_id(0)

    @pl.when(j == 0)
    def _init():
        acc_ref[...] = jnp.zeros_like(acc_ref)

    acc_ref[...] += jnp.sum(h_ref[...], axis=1)

    @pl.when(j == _NT - 1)
    def _finish():
        pooled = acc_ref[...] * (1.0 / _HIDDEN)
        q = jnp.dot(pooled, w_ref[...], preferred_element_type=jnp.float32)
        m = jnp.abs(q)
        iota = jax.lax.broadcasted_iota(jnp.int32, (_B, _D), 1)
        kio = jax.lax.broadcasted_iota(jnp.int32, (_B, _K), 1)

        def step(k, carry):
            m_c, idxs, vals = carry
            mmax = jnp.max(m_c, axis=1, keepdims=True)
            hit = m_c == mmax
            sel_idx = jnp.min(jnp.where(hit, iota, _D), axis=1, keepdims=True)
            sel = iota == sel_idx
            v = jnp.sum(jnp.where(sel, q, 0.0), axis=1, keepdims=True)
            m_c = jnp.where(sel, -1.0, m_c)
            idxs = jnp.where(kio == k, sel_idx, idxs)
            vals = jnp.where(kio == k, v, vals)
            return m_c, idxs, vals

        _, idxs, vals = jax.lax.fori_loop(
            0, _K, step,
            (m,
             jnp.zeros((_B, _K), jnp.int32),
             jnp.zeros((_B, _K), jnp.float32)),
        )
        l1 = jnp.sum(jnp.abs(vals), axis=1, keepdims=True)
        eps = jnp.finfo(jnp.float32).eps
        idx_ref[...] = idxs
        val_ref[...] = vals / jnp.maximum(l1, eps)


def kernel(H_t, W):
    idx, val = pl.pallas_call(
        _fused_body,
        grid=(_NT,),
        in_specs=[
            pl.BlockSpec((_B, _TC, _HIDDEN), lambda j: (0, j, 0)),
            pl.BlockSpec((_HIDDEN, _D), lambda j: (0, 0)),
        ],
        out_specs=[
            pl.BlockSpec((_B, _K), lambda j: (0, 0)),
            pl.BlockSpec((_B, _K), lambda j: (0, 0)),
        ],
        out_shape=[
            jax.ShapeDtypeStruct((_B, _K), jnp.int32),
            jax.ShapeDtypeStruct((_B, _K), jnp.float32),
        ],
        scratch_shapes=[pltpu.VMEM((_B, _D), jnp.float32)],
    )(H_t, W)
    return idx, val


# speedup vs baseline: 1.0401x; 1.0401x over previous
import jax
import jax.numpy as jnp
from jax.experimental import pallas as pl
from jax.experimental.pallas import tpu as pltpu

_B = 4
_D = 2048
_HIDDEN = 2048
_T = 2048
_K = 64

_CH = 512
_NCH = (_B * _T) // _CH
_NBUF = 8
_PER_B = _NCH // _B


def _topk_tail(q, idx_ref, val_ref):
    iota = jax.lax.broadcasted_iota(jnp.int32, (_B, _D), 1)
    kio = jax.lax.broadcasted_iota(jnp.int32, (_B, _K), 1)

    def step(k, carry):
        sq, idxs, vals = carry
        m = jnp.abs(sq)
        mmax = jnp.max(m, axis=1, keepdims=True)
        hit = m == mmax
        sel_idx = jnp.min(jnp.where(hit, iota, _D), axis=1, keepdims=True)
        sel = iota == sel_idx
        v = jnp.sum(jnp.where(sel, sq, 0.0), axis=1, keepdims=True)
        sq = jnp.where(sel, 0.0, sq)
        idxs = jnp.where(kio == k, sel_idx, idxs)
        vals = jnp.where(kio == k, v, vals)
        return sq, idxs, vals

    _, idxs, vals = jax.lax.fori_loop(
        0, _K, step,
        (q,
         jnp.zeros((_B, _K), jnp.int32),
         jnp.zeros((_B, _K), jnp.float32)),
    )
    l1 = jnp.sum(jnp.abs(vals), axis=1, keepdims=True)
    eps = jnp.finfo(jnp.float32).eps
    idx_ref[...] = idxs
    val_ref[...] = vals / jnp.maximum(l1, eps)


def _fused_body(h_hbm, w_hbm, idx_ref, val_ref, wbuf, bufs, acc_ref,
                hsems, wsem):
    pltpu.make_async_copy(w_hbm, wbuf, wsem).start()
    for s in range(_NBUF):
        pltpu.make_async_copy(
            h_hbm.at[pl.ds(s * _CH, _CH)], bufs.at[s], hsems.at[s]).start()

    for i in range(_NCH):
        s = i % _NBUF
        pltpu.make_async_copy(
            h_hbm.at[pl.ds(i * _CH, _CH)], bufs.at[s], hsems.at[s]).wait()
        part = jnp.sum(bufs[s], axis=0, keepdims=True)
        b = i // _PER_B
        if i % _PER_B == 0:
            acc_ref[b:b + 1, :] = part
        else:
            acc_ref[b:b + 1, :] += part
        nxt = i + _NBUF
        if nxt < _NCH:
            pltpu.make_async_copy(
                h_hbm.at[pl.ds(nxt * _CH, _CH)], bufs.at[s],
                hsems.at[s]).start()

    pltpu.make_async_copy(w_hbm, wbuf, wsem).wait()
    pooled = acc_ref[...] * (1.0 / _T)
    q = jnp.dot(pooled, wbuf[...], preferred_element_type=jnp.float32)
    _topk_tail(q, idx_ref, val_ref)


def kernel(H_t, W):
    h_flat = H_t.reshape(_B * _T, _HIDDEN)
    idx, val = pl.pallas_call(
        _fused_body,
        in_specs=[
            pl.BlockSpec(memory_space=pl.ANY),
            pl.BlockSpec(memory_space=pl.ANY),
        ],
        out_specs=[
            pl.BlockSpec((_B, _K), lambda: (0, 0)),
            pl.BlockSpec((_B, _K), lambda: (0, 0)),
        ],
        out_shape=[
            jax.ShapeDtypeStruct((_B, _K), jnp.int32),
            jax.ShapeDtypeStruct((_B, _K), jnp.float32),
        ],
        scratch_shapes=[
            pltpu.VMEM((_HIDDEN, _D), jnp.float32),
            pltpu.VMEM((_NBUF, _CH, _HIDDEN), jnp.float32),
            pltpu.VMEM((_B, _HIDDEN), jnp.float32),
            pltpu.SemaphoreType.DMA((_NBUF,)),
            pltpu.SemaphoreType.DMA,
        ],
    )(h_flat, W)
    return idx, val
